# baseline (speedup 1.0000x reference)
import functools

import jax
import jax.numpy as jnp
from jax import lax
from jax.experimental import pallas as pl
from jax.experimental.pallas import tpu as pltpu
from jax.experimental.pallas import tpu_sc as plsc

_N = 10000
_NP = 10112
_H = 128
_C = 40
_W1 = 160
_W3 = 64
_BLK = 1264
_GRID = _NP // _BLK
_CHUNK = 128
_GSZ = 4
_NSUB = 16
_NCORES = 2
_NTILES = _NCORES * _NSUB
_SLAB = _NP // _NSUB


def _seg_sum(width, n_groups):
  mesh = plsc.VectorSubcoreMesh(
      core_axis_name="c", subcore_axis_name="s",
      num_cores=_NCORES, num_subcores=_NSUB)

  @functools.partial(
      pl.kernel,
      out_type=jax.ShapeDtypeStruct((_NCORES, _NP, width), jnp.bfloat16),
      mesh=mesh,
      compiler_params=pltpu.CompilerParams(use_tc_tiling_on_sc=False),
      scratch_types=[
          pltpu.VMEM((2, _GSZ, _CHUNK), jnp.int32),
          pltpu.VMEM((2, _GSZ, _CHUNK), jnp.int32),
          pltpu.VMEM((2, _CHUNK, width), jnp.bfloat16),
          pltpu.VMEM_SHARED((_NP, width), jnp.bfloat16),
          pltpu.VMEM_SHARED((_NP, width), jnp.bfloat16),
          pltpu.SemaphoreType.DMA,
          pltpu.SemaphoreType.DMA,
          pltpu.SemaphoreType.DMA,
          pltpu.SemaphoreType.DMA,
          pltpu.SemaphoreType.DMA,
      ],
  )
  def seg(table, edges, out, ed, db, buf, acc, tbl, sem_i, sem_a, sem_b,
          sem_sa, sem_sb):
    c = lax.axis_index("c")
    s = lax.axis_index("s")
    w = c * _NSUB + s
    gsem = (sem_a, sem_b)
    ssem = (sem_sa, sem_sb)
    ntail = _SLAB % _CHUNK
    nfull = _SLAB // _CHUNK

    cp_e = pltpu.async_copy(edges.at[w, pl.ds(0, _GSZ)], ed.at[0], sem_i)

    pltpu.sync_copy(table.at[pl.ds(s * _SLAB, _SLAB)],
                    tbl.at[pl.ds(s * _SLAB, _SLAB)])

    zv = jnp.zeros((32,), jnp.bfloat16)

    @pl.loop(0, _CHUNK)
    def _(r):
      for j in range(width // 32):
        buf[0, r, pl.ds(j * 32, 32)] = zv

    for kk in range(nfull):
      pltpu.sync_copy(buf.at[0],
                      acc.at[pl.ds(s * _SLAB + kk * _CHUNK, _CHUNK)])
    if ntail:
      pltpu.sync_copy(buf.at[0, pl.ds(0, ntail)],
                      acc.at[pl.ds(s * _SLAB + nfull * _CHUNK, ntail)])

    def unpack(h):
      for r in range(_GSZ):
        for j in range(_CHUNK // 16):
          v = ed[h, r, pl.ds(j * 16, 16)]
          ed[h, r, pl.ds(j * 16, 16)] = lax.shift_right_logical(v, 14)
          db[h, r, pl.ds(j * 16, 16)] = lax.bitwise_and(v, 16383)

    cp_e.wait()
    unpack(0)
    pltpu.async_copy(edges.at[w, pl.ds(_GSZ, _GSZ)], ed.at[1], sem_i)
    plsc.subcore_barrier()
    pltpu.async_copy(tbl.at[ed.at[0, 0]], buf.at[0], sem_a)

    @pl.loop(0, n_groups // 2)
    def _(i):
      for half in range(2):
        g = 2 * i + half
        not_last = g < n_groups - 1

        @pl.when(not_last)
        def _():
          pltpu.make_async_copy(
              edges.at[w, pl.ds((g + 1) * _GSZ, _GSZ)], ed.at[1 - half],
              sem_i).wait()
          unpack(1 - half)

        for cc in range(_GSZ):
          p = cc % 2
          gsrc = table if cc == _GSZ - 1 else tbl
          pltpu.make_async_copy(
              gsrc.at[ed.at[half, cc]], buf.at[p], gsem[p]).wait()
          pltpu.async_copy(buf.at[p], acc.at[db.at[half, cc]], ssem[p],
                           add=True)
          if half == 0 and cc == 0:
            @pl.when(i > 0)
            def _():
              pltpu.make_async_copy(buf.at[1], acc.at[db.at[half, cc]],
                                    ssem[1]).wait()
          else:
            pltpu.make_async_copy(buf.at[1 - p], acc.at[db.at[half, cc]],
                                  ssem[1 - p]).wait()
          if cc < _GSZ - 1:
            nsrc = table if cc + 1 == _GSZ - 1 else tbl
            pltpu.async_copy(nsrc.at[ed.at[half, cc + 1]],
                             buf.at[1 - p], gsem[1 - p])
          else:
            @pl.when(not_last)
            def _():
              pltpu.async_copy(tbl.at[ed.at[1 - half, 0]],
                               buf.at[0], gsem[0])

        @pl.when(g < n_groups - 2)
        def _():
          pltpu.async_copy(edges.at[w, pl.ds((g + 2) * _GSZ, _GSZ)],
                           ed.at[half], sem_i)

    pltpu.make_async_copy(buf.at[1], acc.at[db.at[1, _GSZ - 1]],
                          ssem[1]).wait()
    plsc.subcore_barrier()
    for kk in range(nfull):
      base = s * _SLAB + kk * _CHUNK
      pltpu.sync_copy(acc.at[pl.ds(base, _CHUNK)], buf.at[0])
      pltpu.sync_copy(buf.at[0], out.at[c, pl.ds(base, _CHUNK)])
    if ntail:
      base = s * _SLAB + nfull * _CHUNK
      pltpu.sync_copy(acc.at[pl.ds(base, ntail)], buf.at[0, pl.ds(0, ntail)])
      pltpu.sync_copy(buf.at[0, pl.ds(0, ntail)],
                      out.at[c, pl.ds(base, ntail)])

  return seg


def _tc_pre(x_ref, wl_ref, wr_ref, e_ref, t_ref, r_ref):
  xb = x_ref[...]
  t_ref[...] = (jnp.dot(xb, wl_ref[...], preferred_element_type=jnp.float32)
                + e_ref[...]).astype(jnp.bfloat16)
  r_ref[...] = jnp.dot(xb, wr_ref[...], preferred_element_type=jnp.float32)


def _tc_mid1(s_ref, r_ref, wl_ref, wr_ref, sc_ref, c_ref,
             p_ref, rout_ref, inv_ref):
  ssum = (s_ref[0].astype(jnp.float32) + s_ref[1].astype(jnp.float32))
  inv = 1.0 / jnp.maximum(ssum[:, _H:_H + 1], 1.0)
  h = jnp.maximum((ssum[:, :_H] * inv + r_ref[...]) * sc_ref[...] + c_ref[...],
                  0.0)
  p_ref[...] = jnp.dot(h, wl_ref[...],
                       preferred_element_type=jnp.float32).astype(jnp.bfloat16)
  rout_ref[...] = jnp.dot(h, wr_ref[...], preferred_element_type=jnp.float32)
  inv_ref[...] = jnp.broadcast_to(inv, (_BLK, 8))


def _tc_mid2(s_ref, r_ref, inv_ref, wl_ref, wr_ref, sc_ref, c_ref, c3_ref,
             p_ref, rout_ref):
  ssum = (s_ref[0].astype(jnp.float32) + s_ref[1].astype(jnp.float32))
  inv = inv_ref[:, 0:1]
  h = jnp.maximum((ssum * inv + r_ref[...]) * sc_ref[...] + c_ref[...], 0.0)
  p_ref[...] = jnp.dot(h, wl_ref[...],
                       preferred_element_type=jnp.float32).astype(jnp.bfloat16)
  rout_ref[...] = jnp.dot(h, wr_ref[...],
                          preferred_element_type=jnp.float32) + c3_ref[...]


def _tc_fin(s_ref, r_ref, inv_ref, o_ref):
  ssum = (s_ref[0].astype(jnp.float32) + s_ref[1].astype(jnp.float32))
  o_ref[...] = ssum * inv_ref[:, 0:1] + r_ref[...]


def _row_spec(width):
  return pl.BlockSpec((_BLK, width), lambda i: (i, 0))


def _full_spec(shape):
  nd = len(shape)
  return pl.BlockSpec(shape, lambda i, _nd=nd: (0,) * _nd)


def _sum_spec(width):
  return pl.BlockSpec((_NCORES, _BLK, width), lambda i: (0, i, 0))


def kernel(x, edge_index, Wl1, bl1, Wr1, g1, b1, Wl2, bl2, Wr2, g2, b2,
           Wl3, bl3, Wr3):
  e = edge_index.shape[1]
  per_group = _NTILES * _CHUNK * _GSZ
  n_groups = -(-e // per_group)
  n_groups += n_groups % 2
  epad = n_groups * per_group
  k_chunks = n_groups * _GSZ

  packed = edge_index[0] * 16384 + edge_index[1]
  pad_idx = jnp.full((epad - e,), _N * 16384 + _N, jnp.int32)
  edges = jnp.concatenate([packed, pad_idx]).reshape(
      _NTILES, k_chunks, _CHUNK)

  xp = jnp.pad(x, ((0, _NP - _N), (0, 0)))
  isq = 1.0 / jnp.sqrt(jnp.float32(1.0 + 1e-5))
  sc1 = (g1 * isq).reshape(1, _H)
  cc1 = (bl1 * g1 * isq + b1).reshape(1, _H)
  sc2 = (g2 * isq).reshape(1, _H)
  cc2 = (bl2 * g2 * isq + b2).reshape(1, _H)

  wl1t = jnp.pad(Wl1.T, ((0, 0), (0, _W1 - _H)))
  e1 = jnp.zeros((1, _W1), jnp.float32).at[0, _H].set(1.0)
  wr1t = Wr1.T
  wl2t = Wl2.T
  wr2t = Wr2.T
  wl3t = jnp.pad(Wl3.T, ((0, 0), (0, _W3 - _C)))
  wr3t = jnp.pad(Wr3.T, ((0, 0), (0, _W3 - _C)))
  c3 = jnp.pad(bl3, (0, _W3 - _C)).reshape(1, _W3)

  t1, r1 = pl.pallas_call(
      _tc_pre,
      grid=(_GRID,),
      in_specs=[_row_spec(_H), _full_spec((_H, _W1)), _full_spec((_H, _H)),
                _full_spec((1, _W1))],
      out_specs=[_row_spec(_W1), _row_spec(_H)],
      out_shape=[jax.ShapeDtypeStruct((_NP, _W1), jnp.bfloat16),
                 jax.ShapeDtypeStruct((_NP, _H), jnp.float32)],
  )(xp, wl1t, wr1t, e1)

  s1 = _seg_sum(_W1, n_groups)(t1, edges)

  p2, r2, inv8 = pl.pallas_call(
      _tc_mid1,
      grid=(_GRID,),
      in_specs=[_sum_spec(_W1), _row_spec(_H), _full_spec((_H, _H)),
                _full_spec((_H, _H)), _full_spec((1, _H)),
                _full_spec((1, _H))],
      out_specs=[_row_spec(_H), _row_spec(_H), _row_spec(8)],
      out_shape=[jax.ShapeDtypeStruct((_NP, _H), jnp.bfloat16),
                 jax.ShapeDtypeStruct((_NP, _H), jnp.float32),
                 jax.ShapeDtypeStruct((_NP, 8), jnp.float32)],
  )(s1, r1, wl2t, wr2t, sc1, cc1)

  s2 = _seg_sum(_H, n_groups)(p2, edges)

  p3, r3 = pl.pallas_call(
      _tc_mid2,
      grid=(_GRID,),
      in_specs=[_sum_spec(_H), _row_spec(_H), _row_spec(8),
                _full_spec((_H, _W3)), _full_spec((_H, _W3)),
                _full_spec((1, _H)), _full_spec((1, _H)),
                _full_spec((1, _W3))],
      out_specs=[_row_spec(_W3), _row_spec(_W3)],
      out_shape=[jax.ShapeDtypeStruct((_NP, _W3), jnp.bfloat16),
                 jax.ShapeDtypeStruct((_NP, _W3), jnp.float32)],
  )(s2, r2, inv8, wl3t, wr3t, sc2, cc2, c3)

  s3 = _seg_sum(_W3, n_groups)(p3, edges)

  o = pl.pallas_call(
      _tc_fin,
      grid=(_GRID,),
      in_specs=[_sum_spec(_W3), _row_spec(_W3), _row_spec(8)],
      out_specs=_row_spec(_W3),
      out_shape=jax.ShapeDtypeStruct((_NP, _W3), jnp.float32),
  )(s3, r3, inv8)

  return o[:_N, :_C]


# speedup vs baseline: 1.3393x; 1.3393x over previous
import functools

import jax
import jax.numpy as jnp
from jax import lax
from jax.experimental import pallas as pl
from jax.experimental.pallas import tpu as pltpu
from jax.experimental.pallas import tpu_sc as plsc

_N = 10000
_NP = 10112
_H = 128
_C = 40
_W1 = 144
_W3 = 64
_BLK = 1264
_GRID = _NP // _BLK
_CHUNK = 128
_GSZ = 4
_NSUB = 16
_NCORES = 2
_NTILES = _NCORES * _NSUB
_SLAB = _NP // _NSUB


def _seg_sum(width, n_groups):
  mesh = plsc.VectorSubcoreMesh(
      core_axis_name="c", subcore_axis_name="s",
      num_cores=_NCORES, num_subcores=_NSUB)

  @functools.partial(
      pl.kernel,
      out_type=jax.ShapeDtypeStruct((_NCORES, _NP, width), jnp.bfloat16),
      mesh=mesh,
      compiler_params=pltpu.CompilerParams(use_tc_tiling_on_sc=False),
      scratch_types=[
          pltpu.VMEM((2, _GSZ, _CHUNK), jnp.int32),
          pltpu.VMEM((2, _GSZ, _CHUNK), jnp.int32),
          pltpu.VMEM((2, _CHUNK, width), jnp.bfloat16),
          pltpu.VMEM_SHARED((_NP, width), jnp.bfloat16),
          pltpu.VMEM_SHARED((_NP, width), jnp.bfloat16),
          pltpu.SemaphoreType.DMA,
          pltpu.SemaphoreType.DMA,
          pltpu.SemaphoreType.DMA,
          pltpu.SemaphoreType.DMA,
          pltpu.SemaphoreType.DMA,
      ],
  )
  def seg(table, edges, out, ed, db, buf, acc, tbl, sem_i, sem_a, sem_b,
          sem_sa, sem_sb):
    c = lax.axis_index("c")
    s = lax.axis_index("s")
    w = c * _NSUB + s
    gsem = (sem_a, sem_b)
    ssem = (sem_sa, sem_sb)
    ntail = _SLAB % _CHUNK
    nfull = _SLAB // _CHUNK

    cp_e = pltpu.async_copy(edges.at[w, pl.ds(0, _GSZ)], ed.at[0], sem_i)

    pltpu.sync_copy(table.at[pl.ds(s * _SLAB, _SLAB)],
                    tbl.at[pl.ds(s * _SLAB, _SLAB)])

    zv = jnp.zeros((32,), jnp.bfloat16)

    @pl.loop(0, _CHUNK)
    def _(r):
      for j in range(width // 32):
        buf[0, r, pl.ds(j * 32, 32)] = zv

    if width % 32:
      zv2 = jnp.zeros((2, 16), jnp.bfloat16)

      @pl.loop(0, _CHUNK // 2)
      def _(r):
        buf[0, pl.ds(2 * r, 2), pl.ds(width - 16, 16)] = zv2

    for kk in range(nfull):
      pltpu.sync_copy(buf.at[0],
                      acc.at[pl.ds(s * _SLAB + kk * _CHUNK, _CHUNK)])
    if ntail:
      pltpu.sync_copy(buf.at[0, pl.ds(0, ntail)],
                      acc.at[pl.ds(s * _SLAB + nfull * _CHUNK, ntail)])

    def unpack(h):
      for r in range(_GSZ):
        for j in range(_CHUNK // 16):
          v = ed[h, r, pl.ds(j * 16, 16)]
          ed[h, r, pl.ds(j * 16, 16)] = lax.shift_right_logical(v, 14)
          db[h, r, pl.ds(j * 16, 16)] = lax.bitwise_and(v, 16383)

    cp_e.wait()
    unpack(0)
    pltpu.async_copy(edges.at[w, pl.ds(_GSZ, _GSZ)], ed.at[1], sem_i)
    plsc.subcore_barrier()
    pltpu.async_copy(tbl.at[ed.at[0, 0]], buf.at[0], sem_a)

    @pl.loop(0, n_groups // 2)
    def _(i):
      for half in range(2):
        g = 2 * i + half
        not_last = g < n_groups - 1

        @pl.when(not_last)
        def _():
          pltpu.make_async_copy(
              edges.at[w, pl.ds((g + 1) * _GSZ, _GSZ)], ed.at[1 - half],
              sem_i).wait()
          unpack(1 - half)

        for cc in range(_GSZ):
          p = cc % 2
          pltpu.make_async_copy(
              tbl.at[ed.at[half, cc]], buf.at[p], gsem[p]).wait()
          pltpu.async_copy(buf.at[p], acc.at[db.at[half, cc]], ssem[p],
                           add=True)
          if half == 0 and cc == 0:
            @pl.when(i > 0)
            def _():
              pltpu.make_async_copy(buf.at[1], acc.at[db.at[half, cc]],
                                    ssem[1]).wait()
          else:
            pltpu.make_async_copy(buf.at[1 - p], acc.at[db.at[half, cc]],
                                  ssem[1 - p]).wait()
          if cc < _GSZ - 1:
            pltpu.async_copy(tbl.at[ed.at[half, cc + 1]],
                             buf.at[1 - p], gsem[1 - p])
          else:
            @pl.when(not_last)
            def _():
              pltpu.async_copy(tbl.at[ed.at[1 - half, 0]],
                               buf.at[0], gsem[0])

        @pl.when(g < n_groups - 2)
        def _():
          pltpu.async_copy(edges.at[w, pl.ds((g + 2) * _GSZ, _GSZ)],
                           ed.at[half], sem_i)

    pltpu.make_async_copy(buf.at[1], acc.at[db.at[1, _GSZ - 1]],
                          ssem[1]).wait()
    plsc.subcore_barrier()
    pltpu.sync_copy(acc.at[pl.ds(s * _SLAB, _SLAB)],
                    out.at[c, pl.ds(s * _SLAB, _SLAB)])

  return seg


def _tc_pre(x_ref, wl_ref, wr_ref, e_ref, t_ref, r_ref):
  xb = x_ref[...]
  t_ref[...] = (jnp.dot(xb, wl_ref[...], preferred_element_type=jnp.float32)
                + e_ref[...]).astype(jnp.bfloat16)
  r_ref[...] = jnp.dot(xb, wr_ref[...], preferred_element_type=jnp.float32)


def _tc_mid1(s_ref, r_ref, wl_ref, wr_ref, sc_ref, c_ref,
             p_ref, rout_ref, inv_ref):
  ssum = (s_ref[0].astype(jnp.float32) + s_ref[1].astype(jnp.float32))
  inv = 1.0 / jnp.maximum(ssum[:, _H:_H + 1], 1.0)
  h = jnp.maximum((ssum[:, :_H] * inv + r_ref[...]) * sc_ref[...] + c_ref[...],
                  0.0)
  p_ref[...] = jnp.dot(h, wl_ref[...],
                       preferred_element_type=jnp.float32).astype(jnp.bfloat16)
  rout_ref[...] = jnp.dot(h, wr_ref[...], preferred_element_type=jnp.float32)
  inv_ref[...] = jnp.broadcast_to(inv, (_BLK, 8))


def _tc_mid2(s_ref, r_ref, inv_ref, wl_ref, wr_ref, sc_ref, c_ref, c3_ref,
             p_ref, rout_ref):
  ssum = (s_ref[0].astype(jnp.float32) + s_ref[1].astype(jnp.float32))
  inv = inv_ref[:, 0:1]
  h = jnp.maximum((ssum * inv + r_ref[...]) * sc_ref[...] + c_ref[...], 0.0)
  p_ref[...] = jnp.dot(h, wl_ref[...],
                       preferred_element_type=jnp.float32).astype(jnp.bfloat16)
  rout_ref[...] = jnp.dot(h, wr_ref[...],
                          preferred_element_type=jnp.float32) + c3_ref[...]


def _tc_fin(s_ref, r_ref, inv_ref, o_ref):
  ssum = (s_ref[0].astype(jnp.float32) + s_ref[1].astype(jnp.float32))
  o_ref[...] = ssum * inv_ref[:, 0:1] + r_ref[...]


def _row_spec(width):
  return pl.BlockSpec((_BLK, width), lambda i: (i, 0))


def _full_spec(shape):
  nd = len(shape)
  return pl.BlockSpec(shape, lambda i, _nd=nd: (0,) * _nd)


def _sum_spec(width):
  return pl.BlockSpec((_NCORES, _BLK, width), lambda i: (0, i, 0))


def kernel(x, edge_index, Wl1, bl1, Wr1, g1, b1, Wl2, bl2, Wr2, g2, b2,
           Wl3, bl3, Wr3):
  e = edge_index.shape[1]
  per_group = _NTILES * _CHUNK * _GSZ
  n_groups = -(-e // per_group)
  n_groups += n_groups % 2
  epad = n_groups * per_group
  k_chunks = n_groups * _GSZ

  packed = edge_index[0] * 16384 + edge_index[1]
  pad_idx = jnp.full((epad - e,), _N * 16384 + _N, jnp.int32)
  edges = jnp.concatenate([packed, pad_idx]).reshape(
      _NTILES, k_chunks, _CHUNK)

  xp = jnp.pad(x, ((0, _NP - _N), (0, 0)))
  isq = 1.0 / jnp.sqrt(jnp.float32(1.0 + 1e-5))
  sc1 = (g1 * isq).reshape(1, _H)
  cc1 = (bl1 * g1 * isq + b1).reshape(1, _H)
  sc2 = (g2 * isq).reshape(1, _H)
  cc2 = (bl2 * g2 * isq + b2).reshape(1, _H)

  wl1t = jnp.pad(Wl1.T, ((0, 0), (0, _W1 - _H)))
  e1 = jnp.zeros((1, _W1), jnp.float32).at[0, _H].set(1.0)
  wr1t = Wr1.T
  wl2t = Wl2.T
  wr2t = Wr2.T
  wl3t = jnp.pad(Wl3.T, ((0, 0), (0, _W3 - _C)))
  wr3t = jnp.pad(Wr3.T, ((0, 0), (0, _W3 - _C)))
  c3 = jnp.pad(bl3, (0, _W3 - _C)).reshape(1, _W3)

  t1, r1 = pl.pallas_call(
      _tc_pre,
      grid=(_GRID,),
      in_specs=[_row_spec(_H), _full_spec((_H, _W1)), _full_spec((_H, _H)),
                _full_spec((1, _W1))],
      out_specs=[_row_spec(_W1), _row_spec(_H)],
      out_shape=[jax.ShapeDtypeStruct((_NP, _W1), jnp.bfloat16),
                 jax.ShapeDtypeStruct((_NP, _H), jnp.float32)],
  )(xp, wl1t, wr1t, e1)

  s1 = _seg_sum(_W1, n_groups)(t1, edges)

  p2, r2, inv8 = pl.pallas_call(
      _tc_mid1,
      grid=(_GRID,),
      in_specs=[_sum_spec(_W1), _row_spec(_H), _full_spec((_H, _H)),
                _full_spec((_H, _H)), _full_spec((1, _H)),
                _full_spec((1, _H))],
      out_specs=[_row_spec(_H), _row_spec(_H), _row_spec(8)],
      out_shape=[jax.ShapeDtypeStruct((_NP, _H), jnp.bfloat16),
                 jax.ShapeDtypeStruct((_NP, _H), jnp.float32),
                 jax.ShapeDtypeStruct((_NP, 8), jnp.float32)],
  )(s1, r1, wl2t, wr2t, sc1, cc1)

  s2 = _seg_sum(_H, n_groups)(p2, edges)

  p3, r3 = pl.pallas_call(
      _tc_mid2,
      grid=(_GRID,),
      in_specs=[_sum_spec(_H), _row_spec(_H), _row_spec(8),
                _full_spec((_H, _W3)), _full_spec((_H, _W3)),
                _full_spec((1, _H)), _full_spec((1, _H)),
                _full_spec((1, _W3))],
      out_specs=[_row_spec(_W3), _row_spec(_W3)],
      out_shape=[jax.ShapeDtypeStruct((_NP, _W3), jnp.bfloat16),
                 jax.ShapeDtypeStruct((_NP, _W3), jnp.float32)],
  )(s2, r2, inv8, wl3t, wr3t, sc2, cc2, c3)

  s3 = _seg_sum(_W3, n_groups)(p3, edges)

  o = pl.pallas_call(
      _tc_fin,
      grid=(_GRID,),
      in_specs=[_sum_spec(_W3), _row_spec(_W3), _row_spec(8)],
      out_specs=_row_spec(_W3),
      out_shape=jax.ShapeDtypeStruct((_NP, _W3), jnp.float32),
  )(s3, r3, inv8)

  return o[:_N, :_C]
